# baseline (speedup 1.0000x reference)
import functools

import jax
import jax.numpy as jnp
from jax import lax
from jax.experimental import pallas as pl
from jax.experimental.pallas import tpu as pltpu
from jax.experimental.pallas import tpu_sc as plsc

_SOFT_DIM = 64
_B = 4
_N = 131072
_NW = 32
_PW = _N // _NW
_SB = 1024
_NV = _SB // 16


def _point_bins(z_v, j):
    q = j * 16
    zv = z_v[pl.ds(q, 16)]
    zc = jnp.minimum(jnp.maximum(zv, -1.0), 1.0)
    zn = (zc + 1.0) / 2.0 * (_SOFT_DIM - 1.0)
    fi = zn.astype(jnp.int32)
    frac = zn - fi.astype(jnp.float32)
    ci = jnp.minimum(fi + 1, _SOFT_DIM - 1)
    qv = lax.iota(jnp.int32, 16) + q
    return fi * _SB + qv, ci * _SB + qv, frac


def _sc_body(z_hbm, out_hbm, z_v, out_v, sem):
    c = lax.axis_index("c")
    s = lax.axis_index("s")
    wid = s * 2 + c

    zeros = jnp.zeros((16,), jnp.float32)

    def zero_vec(j, _):
        out_v[pl.ds(j * 16, 16)] = zeros
        return 0

    lax.fori_loop(0, _SOFT_DIM * _NV, zero_vec, 0)

    for b in range(_B):
        for t in range(_PW // _SB):
            base = wid * _PW + t * _SB

            pltpu.sync_copy(z_hbm.at[b, 0, pl.ds(base, _SB)], z_v)

            def scatter_vals(j, _):
                fidx, cidx, frac = _point_bins(z_v, j)
                plsc.store_scatter(out_v, [cidx], frac)
                plsc.store_scatter(out_v, [fidx], 1.0 - frac)
                return 0

            lax.fori_loop(0, _NV, scatter_vals, 0)

            copies = [
                pltpu.async_copy(
                    out_v.at[pl.ds(d * _SB, _SB)],
                    out_hbm.at[b, d, pl.ds(base, _SB)],
                    sem,
                )
                for d in range(_SOFT_DIM)
            ]
            for cp in copies:
                cp.wait()

            def scatter_zeros(j, _):
                fidx, cidx, _frac = _point_bins(z_v, j)
                plsc.store_scatter(out_v, [cidx], zeros)
                plsc.store_scatter(out_v, [fidx], zeros)
                return 0

            lax.fori_loop(0, _NV, scatter_zeros, 0)


def kernel(z):
    mesh = plsc.VectorSubcoreMesh(core_axis_name="c", subcore_axis_name="s")
    k = functools.partial(
        pl.kernel,
        mesh=mesh,
        out_type=jax.ShapeDtypeStruct((_B, _SOFT_DIM, _N), jnp.float32),
        scratch_types=[
            pltpu.VMEM((_SB,), jnp.float32),
            pltpu.VMEM((_SOFT_DIM * _SB,), jnp.float32),
            pltpu.SemaphoreType.DMA,
        ],
        compiler_params=pltpu.CompilerParams(use_tc_tiling_on_sc=False, needs_layout_passes=False),
    )(_sc_body)
    return k(z)


# speedup vs baseline: 1.0915x; 1.0915x over previous
import functools

import jax
import jax.numpy as jnp
from jax import lax
from jax.experimental import pallas as pl
from jax.experimental.pallas import tpu as pltpu
from jax.experimental.pallas import tpu_sc as plsc

_SOFT_DIM = 64
_B = 4
_N = 131072
_NW = 32
_PW = _N // _NW
_SB = 512
_NV = _SB // 16
_TPB = _PW // _SB
_NCHUNK = _B * _TPB


def _point_bins(z_v, j):
    q = j * 16
    zv = z_v[pl.ds(q, 16)]
    zc = jnp.minimum(jnp.maximum(zv, -1.0), 1.0)
    zn = (zc + 1.0) / 2.0 * (_SOFT_DIM - 1.0)
    fi = zn.astype(jnp.int32)
    frac = zn - fi.astype(jnp.float32)
    ci = jnp.minimum(fi + 1, _SOFT_DIM - 1)
    qv = lax.iota(jnp.int32, 16) + q
    return fi, ci, qv, frac


def _sc_body(z_hbm, out_hbm, z_v, out_v0, out_v1, sem0, sem1):
    c = lax.axis_index("c")
    s = lax.axis_index("s")
    wid = s * 2 + c

    zeros = jnp.zeros((16,), jnp.float32)
    bufs = (out_v0, out_v1)
    sems = (sem0, sem1)

    def _load_z(chunk):
        b = chunk // _TPB
        t = chunk % _TPB
        base = wid * _PW + t * _SB
        pltpu.sync_copy(z_hbm.at[b, 0, pl.ds(base, _SB)], z_v)
        return b, base

    for buf in bufs:
        for d in range(_SOFT_DIM):
            def zero_vec(j, _, _b=buf, _d=d):
                _b[_d, pl.ds(j * 16, 16)] = zeros
                return 0
            lax.fori_loop(0, _NV, zero_vec, 0)

    pending = [None, None]
    for chunk in range(_NCHUNK):
        pi = chunk % 2
        buf = bufs[pi]

        if pending[pi] is not None:
            pending[pi].wait()
            _load_z(chunk - 2)

            def scatter_zeros(j, _, _b=buf):
                fi, ci, qv, _f = _point_bins(z_v, j)
                plsc.store_scatter(_b, [ci, qv], zeros)
                plsc.store_scatter(_b, [fi, qv], zeros)
                return 0

            lax.fori_loop(0, _NV, scatter_zeros, 0)

        b, base = _load_z(chunk)

        def scatter_vals(j, _, _b=buf):
            fi, ci, qv, frac = _point_bins(z_v, j)
            plsc.store_scatter(_b, [ci, qv], frac)
            plsc.store_scatter(_b, [fi, qv], 1.0 - frac)
            return 0

        lax.fori_loop(0, _NV, scatter_vals, 0)

        pending[pi] = pltpu.async_copy(
            buf, out_hbm.at[b, :, pl.ds(base, _SB)], sems[pi]
        )

    for cp in pending:
        if cp is not None:
            cp.wait()


def kernel(z):
    mesh = plsc.VectorSubcoreMesh(core_axis_name="c", subcore_axis_name="s")
    k = functools.partial(
        pl.kernel,
        mesh=mesh,
        out_type=jax.ShapeDtypeStruct((_B, _SOFT_DIM, _N), jnp.float32),
        scratch_types=[
            pltpu.VMEM((_SB,), jnp.float32),
            pltpu.VMEM((_SOFT_DIM, _SB), jnp.float32),
            pltpu.VMEM((_SOFT_DIM, _SB), jnp.float32),
            pltpu.SemaphoreType.DMA,
            pltpu.SemaphoreType.DMA,
        ],
        compiler_params=pltpu.CompilerParams(
            use_tc_tiling_on_sc=False, needs_layout_passes=False
        ),
    )(_sc_body)
    return k(z)


# speedup vs baseline: 5.1987x; 4.7628x over previous
import jax
import jax.numpy as jnp
from jax.experimental import pallas as pl

_SOFT_DIM = 64


def _depth_norm_block(z_ref, out_ref):
    zb = z_ref[0, 0, :]
    z_norm = (jnp.clip(zb, -1.0, 1.0) + 1.0) / 2.0 * (_SOFT_DIM - 1)
    d = jax.lax.broadcasted_iota(
        jnp.int32, (_SOFT_DIM, zb.shape[0]), 0
    ).astype(jnp.float32)
    out_ref[0] = jnp.maximum(1.0 - jnp.abs(z_norm[None, :] - d), 0.0)


def kernel(z):
    B, _, N = z.shape
    nb = 32768
    out = pl.pallas_call(
        _depth_norm_block,
        grid=(B, N // nb),
        in_specs=[pl.BlockSpec((1, 1, nb), lambda b, n: (b, 0, n))],
        out_specs=pl.BlockSpec((1, _SOFT_DIM, nb), lambda b, n: (b, 0, n)),
        out_shape=jax.ShapeDtypeStruct((B, _SOFT_DIM, N), z.dtype),
    )(z)
    return out
